# baseline (speedup 1.0000x reference)
import functools

import jax
import jax.numpy as jnp
from jax import lax
from jax.experimental import pallas as pl
from jax.experimental.pallas import tpu as pltpu
from jax.experimental.pallas import tpu_sc as plsc

NC = 2
NS = 16
LANES = 16
NW = NC * NS

N_PAD = 10240
ROWS_BLK = 2048
G_GRAPHS = 64

NBUF = 5
PREF = 4


def _leaky(t):
    return jnp.where(t >= 0, t, 0.01 * t)


def _rsqrt16(v):
    i = plsc.bitcast(v, jnp.int32)
    y = plsc.bitcast(jnp.int32(0x5F3759DF) - lax.shift_right_logical(i, 1),
                     jnp.float32)
    for _ in range(3):
        y = y * (1.5 - 0.5 * v * y * y)
    return jnp.where(v > 0, y, 0.0)


def _zero_vmem(ref2d, nrows, feat):
    zero = jnp.zeros((LANES,), jnp.float32)

    def zb(i, carry):
        for j in range(feat // LANES):
            ref2d[i, pl.ds(j * LANES, LANES)] = zero
        return carry

    lax.fori_loop(0, nrows, zb, 0)


def _edge_phase(hp_h, src_v, dst_v, nrm_v, rows_v, acc_sh, gsem, ssem,
                nchunk, K, feat):
    for b in range(PREF):
        pltpu.async_copy(hp_h.at[src_v.at[b]], rows_v.at[b], gsem.at[b])

    def outer(t, carry):
        for b in range(NBUF):
            c = t * NBUF + b
            pltpu.make_async_copy(
                hp_h.at[src_v.at[c]], rows_v.at[b], gsem.at[b]).wait()

            def scale(kq, cc):
                for u in range(4):
                    k = kq * 4 + u
                    wb = plsc.load_gather(
                        nrm_v, [jnp.full((LANES,), c * K + k, jnp.int32)])
                    for j in range(feat // LANES):
                        sl = pl.ds(j * LANES, LANES)
                        rows_v[b, k, sl] = rows_v[b, k, sl] * wb
                return cc

            lax.fori_loop(0, K // 4, scale, 0)
            pltpu.async_copy(rows_v.at[b], acc_sh.at[dst_v.at[c]],
                             ssem.at[b], add=True)
            cf = c + PREF
            bf = (b + PREF) % NBUF

            @pl.when(cf < nchunk)
            def _():
                @pl.when(cf >= NBUF)
                def _():
                    pltpu.make_async_copy(
                        rows_v.at[bf], acc_sh.at[dst_v.at[c]],
                        ssem.at[bf]).wait()

                pltpu.async_copy(
                    hp_h.at[src_v.at[cf]], rows_v.at[bf], gsem.at[bf])

        return carry

    lax.fori_loop(0, nchunk // NBUF, outer, 0)
    for b in range(NBUF):
        pltpu.make_async_copy(
            rows_v.at[b], acc_sh.at[dst_v.at[0]], ssem.at[b]).wait()


def _sc_layer1(h1, src3, dst3, ew2, feat):
    _, nchunk, K = src3.shape
    nper = N_PAD // NS
    ew_per = nchunk * K
    mesh = plsc.VectorSubcoreMesh(core_axis_name="c", subcore_axis_name="s")

    @functools.partial(
        pl.kernel,
        out_type=(jax.ShapeDtypeStruct((NC, N_PAD, feat), jnp.float32),
                  jax.ShapeDtypeStruct((NW, ew_per), jnp.float32)),
        mesh=mesh,
        compiler_params=pltpu.CompilerParams(needs_layout_passes=False, use_tc_tiling_on_sc=False),
        scratch_types=[
            pltpu.VMEM((nchunk, K), jnp.int32),
            pltpu.VMEM((nchunk, K), jnp.int32),
            pltpu.VMEM((ew_per,), jnp.float32),
            pltpu.VMEM((ew_per,), jnp.float32),
            pltpu.VMEM((N_PAD,), jnp.float32),
            pltpu.VMEM((NBUF, K, feat), jnp.float32),
            pltpu.VMEM_SHARED((N_PAD, feat), jnp.float32),
            pltpu.VMEM_SHARED((N_PAD,), jnp.float32),
            pltpu.SemaphoreType.DMA((NBUF,)),
            pltpu.SemaphoreType.DMA((NBUF,)),
            pltpu.SemaphoreType.DMA,
        ],
    )
    def k1(h1_h, src_h, dst_h, ew_h, agg_h, nrm_h,
           src_v, dst_v, ew_v, nrm_v, dinv_v, rows_v,
           acc_sh, deg_sh, gsem, ssem, dsem):
        ci = lax.axis_index("c")
        si = lax.axis_index("s")
        w = ci * NS + si

        zero = jnp.zeros((LANES,), jnp.float32)

        def z1(i, carry):
            nrm_v[pl.ds(i * LANES, LANES)] = zero
            return carry

        lax.fori_loop(0, nper // LANES, z1, 0)
        pltpu.sync_copy(nrm_v.at[pl.ds(0, nper)],
                        deg_sh.at[pl.ds(si * nper, nper)])
        _zero_vmem(rows_v.at[0], K, feat)
        for p in range(nper // K):
            pltpu.sync_copy(rows_v.at[0],
                            acc_sh.at[pl.ds(si * nper + p * K, K)])
        plsc.subcore_barrier()

        for half in range(2):
            wd = si * 2 + half
            pltpu.sync_copy(dst_h.at[wd], dst_v)
            pltpu.sync_copy(ew_h.at[wd], ew_v)

            def fire(c, carry):
                pltpu.async_copy(ew_v.at[pl.ds(c * K, K)],
                                 deg_sh.at[dst_v.at[c]], dsem, add=True)
                return carry

            lax.fori_loop(0, nchunk, fire, 0)

            def drain(c, carry):
                pltpu.make_async_copy(ew_v.at[pl.ds(0, K)],
                                      deg_sh.at[dst_v.at[0]], dsem).wait()
                return carry

            lax.fori_loop(0, nchunk, drain, 0)
        plsc.subcore_barrier()

        pltpu.sync_copy(deg_sh.at[pl.ds(si * nper, nper)],
                        nrm_v.at[pl.ds(0, nper)])

        def binv(i, carry):
            sl = pl.ds(i * LANES, LANES)
            nrm_v[sl] = _rsqrt16(nrm_v[sl])
            return carry

        lax.fori_loop(0, nper // LANES, binv, 0)
        pltpu.sync_copy(nrm_v.at[pl.ds(0, nper)],
                        deg_sh.at[pl.ds(si * nper, nper)])
        plsc.subcore_barrier()
        pltpu.sync_copy(deg_sh, dinv_v)

        pltpu.sync_copy(src_h.at[w], src_v)
        pltpu.sync_copy(dst_h.at[w], dst_v)
        pltpu.sync_copy(ew_h.at[w], ew_v)

        def bnorm(c, carry):
            for j in range(K // LANES):
                sl = pl.ds(j * LANES, LANES)
                s16 = src_v[c, sl]
                d16 = dst_v[c, sl]
                w16 = ew_v[pl.ds(c * K + j * LANES, LANES)]
                nrm = (plsc.load_gather(dinv_v, [s16]) * w16 *
                       plsc.load_gather(dinv_v, [d16]))
                nrm_v[pl.ds(c * K + j * LANES, LANES)] = nrm
            return carry

        lax.fori_loop(0, nchunk, bnorm, 0)
        pltpu.sync_copy(nrm_v, nrm_h.at[w])

        _edge_phase(h1_h, src_v, dst_v, nrm_v, rows_v, acc_sh, gsem, ssem,
                    nchunk, K, feat)
        plsc.subcore_barrier()
        pltpu.sync_copy(acc_sh.at[pl.ds(si * nper, nper)],
                        agg_h.at[ci, pl.ds(si * nper, nper)])

    return k1(h1, src3, dst3, ew2)


def _sc_layer2(hp, src3, dst3, nrm2, feat):
    _, nchunk, K = src3.shape
    nper = N_PAD // NS
    ew_per = nchunk * K
    mesh = plsc.VectorSubcoreMesh(core_axis_name="c", subcore_axis_name="s")

    @functools.partial(
        pl.kernel,
        out_type=jax.ShapeDtypeStruct((NC, N_PAD, feat), jnp.float32),
        mesh=mesh,
        compiler_params=pltpu.CompilerParams(needs_layout_passes=False, use_tc_tiling_on_sc=False),
        scratch_types=[
            pltpu.VMEM((nchunk, K), jnp.int32),
            pltpu.VMEM((nchunk, K), jnp.int32),
            pltpu.VMEM((ew_per,), jnp.float32),
            pltpu.VMEM((NBUF, K, feat), jnp.float32),
            pltpu.VMEM_SHARED((N_PAD, feat), jnp.float32),
            pltpu.SemaphoreType.DMA((NBUF,)),
            pltpu.SemaphoreType.DMA((NBUF,)),
        ],
    )
    def k2(hp_h, src_h, dst_h, nrm_h, out_h,
           src_v, dst_v, nrm_v, rows_v, acc_sh, gsem, ssem):
        ci = lax.axis_index("c")
        si = lax.axis_index("s")
        w = ci * NS + si
        pltpu.sync_copy(src_h.at[w], src_v)
        pltpu.sync_copy(dst_h.at[w], dst_v)
        pltpu.sync_copy(nrm_h.at[w], nrm_v)
        _zero_vmem(rows_v.at[0], K, feat)
        for p in range(nper // K):
            pltpu.sync_copy(rows_v.at[0],
                            acc_sh.at[pl.ds(si * nper + p * K, K)])
        plsc.subcore_barrier()
        _edge_phase(hp_h, src_v, dst_v, nrm_v, rows_v, acc_sh, gsem, ssem,
                    nchunk, K, feat)
        plsc.subcore_barrier()
        pltpu.sync_copy(acc_sh.at[pl.ds(si * nper, nper)],
                        out_h.at[ci, pl.ds(si * nper, nper)])

    return k2(hp, src3, dst3, nrm2)


def _tc_h1(x_p, W1):
    n, d = x_p.shape
    h = W1.shape[0]

    def body(x_ref, w_ref, o_ref):
        o_ref[...] = lax.dot_general(x_ref[...], w_ref[...],
                                     (((1,), (1,)), ((), ())),
                                     preferred_element_type=jnp.float32)

    return pl.pallas_call(
        body,
        grid=(n // ROWS_BLK,),
        in_specs=[
            pl.BlockSpec((ROWS_BLK, d), lambda i: (i, 0)),
            pl.BlockSpec((h, d), lambda i: (0, 0)),
        ],
        out_specs=pl.BlockSpec((ROWS_BLK, h), lambda i: (i, 0)),
        out_shape=jax.ShapeDtypeStruct((n, h), jnp.float32),
    )(x_p, W1)


def _tc_mid(aggp, b1, W2):
    n = aggp.shape[1]
    h = aggp.shape[2]

    def body(agg_ref, b_ref, w_ref, o_ref):
        o1 = _leaky(agg_ref[0] + agg_ref[1] + b_ref[...])
        o_ref[...] = lax.dot_general(o1, w_ref[...], (((1,), (1,)), ((), ())),
                                     preferred_element_type=jnp.float32)

    return pl.pallas_call(
        body,
        grid=(n // ROWS_BLK,),
        in_specs=[
            pl.BlockSpec((NC, ROWS_BLK, h), lambda i: (0, i, 0)),
            pl.BlockSpec((1, h), lambda i: (0, 0)),
            pl.BlockSpec((h, h), lambda i: (0, 0)),
        ],
        out_specs=pl.BlockSpec((ROWS_BLK, h), lambda i: (i, 0)),
        out_shape=jax.ShapeDtypeStruct((n, h), jnp.float32),
    )(aggp, b1, W2)


def _tc_head(aggp, b2, batch2, Wl, bl):
    n = aggp.shape[1]
    h = aggp.shape[2]
    c = Wl.shape[0]
    ngrid = n // ROWS_BLK

    def body(agg_ref, b_ref, batch_ref, wl_ref, bl_ref, o_ref, acc_ref):
        i = pl.program_id(0)
        o2 = _leaky(agg_ref[0] + agg_ref[1] + b_ref[...])
        gids = lax.broadcasted_iota(jnp.int32, (G_GRAPHS, ROWS_BLK), 0)
        onehot = (gids == batch_ref[...]).astype(jnp.float32)
        aug = jnp.concatenate(
            [o2, jnp.ones((ROWS_BLK, 2 * h - h), jnp.float32)], axis=1)
        p = lax.dot_general(onehot, aug, (((1,), (0,)), ((), ())),
                            preferred_element_type=jnp.float32)

        @pl.when(i == 0)
        def _():
            acc_ref[...] = p

        @pl.when(i > 0)
        def _():
            acc_ref[...] = acc_ref[...] + p

        @pl.when(i == ngrid - 1)
        def _():
            acc = acc_ref[...]
            cnt = jnp.maximum(acc[:, h:h + 1], 1.0)
            pooled = acc[:, :h] / cnt
            logits = lax.dot_general(pooled, wl_ref[...], (((1,), (1,)), ((), ())),
                                     preferred_element_type=jnp.float32)
            logits = logits + bl_ref[...]
            m = jnp.max(logits, axis=1, keepdims=True)
            lse = m + jnp.log(jnp.sum(jnp.exp(logits - m), axis=1, keepdims=True))
            o_ref[...] = logits - lse

    return pl.pallas_call(
        body,
        grid=(ngrid,),
        in_specs=[
            pl.BlockSpec((NC, ROWS_BLK, h), lambda i: (0, i, 0)),
            pl.BlockSpec((1, h), lambda i: (0, 0)),
            pl.BlockSpec((1, ROWS_BLK), lambda i: (0, i)),
            pl.BlockSpec((c, h), lambda i: (0, 0)),
            pl.BlockSpec((1, c), lambda i: (0, 0)),
        ],
        out_specs=pl.BlockSpec((G_GRAPHS, c), lambda i: (0, 0)),
        out_shape=jax.ShapeDtypeStruct((G_GRAPHS, c), jnp.float32),
        scratch_shapes=[pltpu.VMEM((G_GRAPHS, 2 * h), jnp.float32)],
    )(aggp, b2, batch2, Wl, bl)


def kernel(x, edge_index, edge_weight, batch, W1, b1, W2, b2, Wl, bl):
    n, d = x.shape
    h = W1.shape[0]
    c = Wl.shape[0]
    e = edge_weight.shape[0]

    ew_per = e // NW
    K = 80
    nchunk = ew_per // K

    src3 = edge_index[0].reshape(NW, nchunk, K)
    dst3 = edge_index[1].reshape(NW, nchunk, K)
    ew2 = edge_weight.reshape(NW, ew_per)
    x_p = jnp.concatenate([x, jnp.zeros((N_PAD - n, d), jnp.float32)], axis=0)
    batch2 = jnp.concatenate(
        [batch, jnp.full((N_PAD - n,), G_GRAPHS, jnp.int32)]).reshape(1, N_PAD)
    b1r = b1.reshape(1, h)
    b2r = b2.reshape(1, h)
    blr = bl.reshape(1, c)

    h1 = _tc_h1(x_p, W1)
    agg1, nrm2 = _sc_layer1(h1, src3, dst3, ew2, h)
    h2 = _tc_mid(agg1, b1r, W2)
    agg2 = _sc_layer2(h2, src3, dst3, nrm2, h)
    return _tc_head(agg2, b2r, batch2, Wl, blr)


# speedup vs baseline: 1.0022x; 1.0022x over previous
import functools

import jax
import jax.numpy as jnp
from jax import lax
from jax.experimental import pallas as pl
from jax.experimental.pallas import tpu as pltpu
from jax.experimental.pallas import tpu_sc as plsc

NC = 2
NS = 16
LANES = 16
NW = NC * NS

N_PAD = 10240
ROWS_BLK = 2048
G_GRAPHS = 64

NBUF = 5
PREF = 3


def _leaky(t):
    return jnp.where(t >= 0, t, 0.01 * t)


def _rsqrt16(v):
    i = plsc.bitcast(v, jnp.int32)
    y = plsc.bitcast(jnp.int32(0x5F3759DF) - lax.shift_right_logical(i, 1),
                     jnp.float32)
    for _ in range(3):
        y = y * (1.5 - 0.5 * v * y * y)
    return jnp.where(v > 0, y, 0.0)


def _zero_vmem(ref2d, nrows, feat):
    zero = jnp.zeros((LANES,), jnp.float32)

    def zb(i, carry):
        for j in range(feat // LANES):
            ref2d[i, pl.ds(j * LANES, LANES)] = zero
        return carry

    lax.fori_loop(0, nrows, zb, 0)


def _edge_phase(hp_h, src_v, dst_v, nrm_v, rows_v, acc_sh, gsem, ssem,
                nchunk, K, feat):
    for b in range(PREF):
        pltpu.async_copy(hp_h.at[src_v.at[b]], rows_v.at[b], gsem.at[b])

    def outer(t, carry):
        for b in range(NBUF):
            c = t * NBUF + b
            pltpu.make_async_copy(
                hp_h.at[src_v.at[c]], rows_v.at[b], gsem.at[b]).wait()

            def scale(kq, cc):
                for u in range(4):
                    k = kq * 4 + u
                    wb = plsc.load_gather(
                        nrm_v, [jnp.full((LANES,), c * K + k, jnp.int32)])
                    for j in range(feat // LANES):
                        sl = pl.ds(j * LANES, LANES)
                        rows_v[b, k, sl] = rows_v[b, k, sl] * wb
                return cc

            lax.fori_loop(0, K // 4, scale, 0)
            pltpu.async_copy(rows_v.at[b], acc_sh.at[dst_v.at[c]],
                             ssem.at[b], add=True)
            cf = c + PREF
            bf = (b + PREF) % NBUF

            @pl.when(cf < nchunk)
            def _():
                @pl.when(cf >= NBUF)
                def _():
                    pltpu.make_async_copy(
                        rows_v.at[bf], acc_sh.at[dst_v.at[c]],
                        ssem.at[bf]).wait()

                pltpu.async_copy(
                    hp_h.at[src_v.at[cf]], rows_v.at[bf], gsem.at[bf])

        return carry

    lax.fori_loop(0, nchunk // NBUF, outer, 0)
    for b in range(NBUF):
        pltpu.make_async_copy(
            rows_v.at[b], acc_sh.at[dst_v.at[0]], ssem.at[b]).wait()


def _sc_layer1(h1, src3, dst3, ew2, feat):
    _, nchunk, K = src3.shape
    nper = N_PAD // NS
    ew_per = nchunk * K
    mesh = plsc.VectorSubcoreMesh(core_axis_name="c", subcore_axis_name="s")

    @functools.partial(
        pl.kernel,
        out_type=(jax.ShapeDtypeStruct((NC, N_PAD, feat), jnp.float32),
                  jax.ShapeDtypeStruct((NW, ew_per), jnp.float32)),
        mesh=mesh,
        compiler_params=pltpu.CompilerParams(needs_layout_passes=False, use_tc_tiling_on_sc=False),
        scratch_types=[
            pltpu.VMEM((nchunk, K), jnp.int32),
            pltpu.VMEM((nchunk, K), jnp.int32),
            pltpu.VMEM((ew_per,), jnp.float32),
            pltpu.VMEM((ew_per,), jnp.float32),
            pltpu.VMEM((N_PAD,), jnp.float32),
            pltpu.VMEM((NBUF, K, feat), jnp.float32),
            pltpu.VMEM_SHARED((N_PAD, feat), jnp.float32),
            pltpu.VMEM_SHARED((N_PAD,), jnp.float32),
            pltpu.SemaphoreType.DMA((NBUF,)),
            pltpu.SemaphoreType.DMA((NBUF,)),
            pltpu.SemaphoreType.DMA,
        ],
    )
    def k1(h1_h, src_h, dst_h, ew_h, agg_h, nrm_h,
           src_v, dst_v, ew_v, nrm_v, dinv_v, rows_v,
           acc_sh, deg_sh, gsem, ssem, dsem):
        ci = lax.axis_index("c")
        si = lax.axis_index("s")
        w = ci * NS + si

        zero = jnp.zeros((LANES,), jnp.float32)

        def z1(i, carry):
            nrm_v[pl.ds(i * LANES, LANES)] = zero
            return carry

        lax.fori_loop(0, nper // LANES, z1, 0)
        pltpu.sync_copy(nrm_v.at[pl.ds(0, nper)],
                        deg_sh.at[pl.ds(si * nper, nper)])
        _zero_vmem(rows_v.at[0], K, feat)
        for p in range(nper // K):
            pltpu.sync_copy(rows_v.at[0],
                            acc_sh.at[pl.ds(si * nper + p * K, K)])
        plsc.subcore_barrier()

        for half in range(2):
            wd = si * 2 + half
            pltpu.sync_copy(dst_h.at[wd], dst_v)
            pltpu.sync_copy(ew_h.at[wd], ew_v)

            def fire(c, carry):
                pltpu.async_copy(ew_v.at[pl.ds(c * K, K)],
                                 deg_sh.at[dst_v.at[c]], dsem, add=True)
                return carry

            lax.fori_loop(0, nchunk, fire, 0)

            def drain(c, carry):
                pltpu.make_async_copy(ew_v.at[pl.ds(0, K)],
                                      deg_sh.at[dst_v.at[0]], dsem).wait()
                return carry

            lax.fori_loop(0, nchunk, drain, 0)
        plsc.subcore_barrier()

        pltpu.sync_copy(deg_sh.at[pl.ds(si * nper, nper)],
                        nrm_v.at[pl.ds(0, nper)])

        def binv(i, carry):
            sl = pl.ds(i * LANES, LANES)
            nrm_v[sl] = _rsqrt16(nrm_v[sl])
            return carry

        lax.fori_loop(0, nper // LANES, binv, 0)
        pltpu.sync_copy(nrm_v.at[pl.ds(0, nper)],
                        deg_sh.at[pl.ds(si * nper, nper)])
        plsc.subcore_barrier()
        pltpu.sync_copy(deg_sh, dinv_v)

        pltpu.sync_copy(src_h.at[w], src_v)
        pltpu.sync_copy(dst_h.at[w], dst_v)
        pltpu.sync_copy(ew_h.at[w], ew_v)

        def bnorm(c, carry):
            for j in range(K // LANES):
                sl = pl.ds(j * LANES, LANES)
                s16 = src_v[c, sl]
                d16 = dst_v[c, sl]
                w16 = ew_v[pl.ds(c * K + j * LANES, LANES)]
                nrm = (plsc.load_gather(dinv_v, [s16]) * w16 *
                       plsc.load_gather(dinv_v, [d16]))
                nrm_v[pl.ds(c * K + j * LANES, LANES)] = nrm
            return carry

        lax.fori_loop(0, nchunk, bnorm, 0)
        pltpu.sync_copy(nrm_v, nrm_h.at[w])

        _edge_phase(h1_h, src_v, dst_v, nrm_v, rows_v, acc_sh, gsem, ssem,
                    nchunk, K, feat)
        plsc.subcore_barrier()
        pltpu.sync_copy(acc_sh.at[pl.ds(si * nper, nper)],
                        agg_h.at[ci, pl.ds(si * nper, nper)])

    return k1(h1, src3, dst3, ew2)


def _sc_layer2(hp, src3, dst3, nrm2, feat):
    _, nchunk, K = src3.shape
    nper = N_PAD // NS
    ew_per = nchunk * K
    mesh = plsc.VectorSubcoreMesh(core_axis_name="c", subcore_axis_name="s")

    @functools.partial(
        pl.kernel,
        out_type=jax.ShapeDtypeStruct((NC, N_PAD, feat), jnp.float32),
        mesh=mesh,
        compiler_params=pltpu.CompilerParams(needs_layout_passes=False, use_tc_tiling_on_sc=False),
        scratch_types=[
            pltpu.VMEM((nchunk, K), jnp.int32),
            pltpu.VMEM((nchunk, K), jnp.int32),
            pltpu.VMEM((ew_per,), jnp.float32),
            pltpu.VMEM((NBUF, K, feat), jnp.float32),
            pltpu.VMEM_SHARED((N_PAD, feat), jnp.float32),
            pltpu.SemaphoreType.DMA((NBUF,)),
            pltpu.SemaphoreType.DMA((NBUF,)),
        ],
    )
    def k2(hp_h, src_h, dst_h, nrm_h, out_h,
           src_v, dst_v, nrm_v, rows_v, acc_sh, gsem, ssem):
        ci = lax.axis_index("c")
        si = lax.axis_index("s")
        w = ci * NS + si
        pltpu.sync_copy(src_h.at[w], src_v)
        pltpu.sync_copy(dst_h.at[w], dst_v)
        pltpu.sync_copy(nrm_h.at[w], nrm_v)
        _zero_vmem(rows_v.at[0], K, feat)
        for p in range(nper // K):
            pltpu.sync_copy(rows_v.at[0],
                            acc_sh.at[pl.ds(si * nper + p * K, K)])
        plsc.subcore_barrier()
        _edge_phase(hp_h, src_v, dst_v, nrm_v, rows_v, acc_sh, gsem, ssem,
                    nchunk, K, feat)
        plsc.subcore_barrier()
        pltpu.sync_copy(acc_sh.at[pl.ds(si * nper, nper)],
                        out_h.at[ci, pl.ds(si * nper, nper)])

    return k2(hp, src3, dst3, nrm2)


def _tc_h1(x_p, W1):
    n, d = x_p.shape
    h = W1.shape[0]

    def body(x_ref, w_ref, o_ref):
        o_ref[...] = lax.dot_general(x_ref[...], w_ref[...],
                                     (((1,), (1,)), ((), ())),
                                     preferred_element_type=jnp.float32)

    return pl.pallas_call(
        body,
        grid=(n // ROWS_BLK,),
        in_specs=[
            pl.BlockSpec((ROWS_BLK, d), lambda i: (i, 0)),
            pl.BlockSpec((h, d), lambda i: (0, 0)),
        ],
        out_specs=pl.BlockSpec((ROWS_BLK, h), lambda i: (i, 0)),
        out_shape=jax.ShapeDtypeStruct((n, h), jnp.float32),
    )(x_p, W1)


def _tc_mid(aggp, b1, W2):
    n = aggp.shape[1]
    h = aggp.shape[2]

    def body(agg_ref, b_ref, w_ref, o_ref):
        o1 = _leaky(agg_ref[0] + agg_ref[1] + b_ref[...])
        o_ref[...] = lax.dot_general(o1, w_ref[...], (((1,), (1,)), ((), ())),
                                     preferred_element_type=jnp.float32)

    return pl.pallas_call(
        body,
        grid=(n // ROWS_BLK,),
        in_specs=[
            pl.BlockSpec((NC, ROWS_BLK, h), lambda i: (0, i, 0)),
            pl.BlockSpec((1, h), lambda i: (0, 0)),
            pl.BlockSpec((h, h), lambda i: (0, 0)),
        ],
        out_specs=pl.BlockSpec((ROWS_BLK, h), lambda i: (i, 0)),
        out_shape=jax.ShapeDtypeStruct((n, h), jnp.float32),
    )(aggp, b1, W2)


def _tc_head(aggp, b2, batch2, Wl, bl):
    n = aggp.shape[1]
    h = aggp.shape[2]
    c = Wl.shape[0]
    ngrid = n // ROWS_BLK

    def body(agg_ref, b_ref, batch_ref, wl_ref, bl_ref, o_ref, acc_ref):
        i = pl.program_id(0)
        o2 = _leaky(agg_ref[0] + agg_ref[1] + b_ref[...])
        gids = lax.broadcasted_iota(jnp.int32, (G_GRAPHS, ROWS_BLK), 0)
        onehot = (gids == batch_ref[...]).astype(jnp.float32)
        aug = jnp.concatenate(
            [o2, jnp.ones((ROWS_BLK, 2 * h - h), jnp.float32)], axis=1)
        p = lax.dot_general(onehot, aug, (((1,), (0,)), ((), ())),
                            preferred_element_type=jnp.float32)

        @pl.when(i == 0)
        def _():
            acc_ref[...] = p

        @pl.when(i > 0)
        def _():
            acc_ref[...] = acc_ref[...] + p

        @pl.when(i == ngrid - 1)
        def _():
            acc = acc_ref[...]
            cnt = jnp.maximum(acc[:, h:h + 1], 1.0)
            pooled = acc[:, :h] / cnt
            logits = lax.dot_general(pooled, wl_ref[...], (((1,), (1,)), ((), ())),
                                     preferred_element_type=jnp.float32)
            logits = logits + bl_ref[...]
            m = jnp.max(logits, axis=1, keepdims=True)
            lse = m + jnp.log(jnp.sum(jnp.exp(logits - m), axis=1, keepdims=True))
            o_ref[...] = logits - lse

    return pl.pallas_call(
        body,
        grid=(ngrid,),
        in_specs=[
            pl.BlockSpec((NC, ROWS_BLK, h), lambda i: (0, i, 0)),
            pl.BlockSpec((1, h), lambda i: (0, 0)),
            pl.BlockSpec((1, ROWS_BLK), lambda i: (0, i)),
            pl.BlockSpec((c, h), lambda i: (0, 0)),
            pl.BlockSpec((1, c), lambda i: (0, 0)),
        ],
        out_specs=pl.BlockSpec((G_GRAPHS, c), lambda i: (0, 0)),
        out_shape=jax.ShapeDtypeStruct((G_GRAPHS, c), jnp.float32),
        scratch_shapes=[pltpu.VMEM((G_GRAPHS, 2 * h), jnp.float32)],
    )(aggp, b2, batch2, Wl, bl)


def kernel(x, edge_index, edge_weight, batch, W1, b1, W2, b2, Wl, bl):
    n, d = x.shape
    h = W1.shape[0]
    c = Wl.shape[0]
    e = edge_weight.shape[0]

    ew_per = e // NW
    K = 80
    nchunk = ew_per // K

    src3 = edge_index[0].reshape(NW, nchunk, K)
    dst3 = edge_index[1].reshape(NW, nchunk, K)
    ew2 = edge_weight.reshape(NW, ew_per)
    x_p = jnp.concatenate([x, jnp.zeros((N_PAD - n, d), jnp.float32)], axis=0)
    batch2 = jnp.concatenate(
        [batch, jnp.full((N_PAD - n,), G_GRAPHS, jnp.int32)]).reshape(1, N_PAD)
    b1r = b1.reshape(1, h)
    b2r = b2.reshape(1, h)
    blr = bl.reshape(1, c)

    h1 = _tc_h1(x_p, W1)
    agg1, nrm2 = _sc_layer1(h1, src3, dst3, ew2, h)
    h2 = _tc_mid(agg1, b1r, W2)
    agg2 = _sc_layer2(h2, src3, dst3, nrm2, h)
    return _tc_head(agg2, b2r, batch2, Wl, blr)


# speedup vs baseline: 1.0173x; 1.0150x over previous
import functools

import jax
import jax.numpy as jnp
from jax import lax
from jax.experimental import pallas as pl
from jax.experimental.pallas import tpu as pltpu
from jax.experimental.pallas import tpu_sc as plsc

NC = 2
NS = 16
LANES = 16
NW = NC * NS

N_PAD = 10240
ROWS_BLK = 2048
G_GRAPHS = 64

NBUF = 5
PREF = 3


def _leaky(t):
    return jnp.where(t >= 0, t, 0.01 * t)


def _rsqrt16(v):
    i = plsc.bitcast(v, jnp.int32)
    y = plsc.bitcast(jnp.int32(0x5F3759DF) - lax.shift_right_logical(i, 1),
                     jnp.float32)
    for _ in range(3):
        y = y * (1.5 - 0.5 * v * y * y)
    return jnp.where(v > 0, y, 0.0)


def _zero_vmem(ref2d, nrows, feat):
    zero = jnp.zeros((LANES,), jnp.float32)

    def zb(i, carry):
        for j in range(feat // LANES):
            ref2d[i, pl.ds(j * LANES, LANES)] = zero
        return carry

    lax.fori_loop(0, nrows, zb, 0)


def _edge_phase(hp_h, src_v, dst_v, nrm_v, rows_v, acc_sh, gsem, ssem,
                nchunk, K, feat):
    for b in range(PREF):
        pltpu.async_copy(hp_h.at[src_v.at[b]], rows_v.at[b], gsem.at[b])

    def outer(t, carry):
        for b in range(NBUF):
            c = t * NBUF + b
            pltpu.make_async_copy(
                hp_h.at[src_v.at[c]], rows_v.at[b], gsem.at[b]).wait()

            def scale(kq, cc):
                for u in range(4):
                    k = kq * 4 + u
                    wb = plsc.load_gather(
                        nrm_v, [jnp.full((LANES,), c * K + k, jnp.int32)])
                    for j in range(feat // LANES):
                        sl = pl.ds(j * LANES, LANES)
                        rows_v[b, k, sl] = rows_v[b, k, sl] * wb
                return cc

            lax.fori_loop(0, K // 4, scale, 0)
            pltpu.async_copy(rows_v.at[b], acc_sh.at[dst_v.at[c]],
                             ssem.at[b], add=True)
            cf = c + PREF
            bf = (b + PREF) % NBUF

            @pl.when(cf < nchunk)
            def _():
                @pl.when(cf >= NBUF)
                def _():
                    pltpu.make_async_copy(
                        rows_v.at[bf], acc_sh.at[dst_v.at[c]],
                        ssem.at[bf]).wait()

                pltpu.async_copy(
                    hp_h.at[src_v.at[cf]], rows_v.at[bf], gsem.at[bf])

        return carry

    lax.fori_loop(0, nchunk // NBUF, outer, 0)
    for b in range(NBUF):
        pltpu.make_async_copy(
            rows_v.at[b], acc_sh.at[dst_v.at[0]], ssem.at[b]).wait()


def _sc_layer1(h1, src3, dst3, ew2, feat):
    _, nchunk, K = src3.shape
    nper = N_PAD // NS
    ew_per = nchunk * K
    mesh = plsc.VectorSubcoreMesh(core_axis_name="c", subcore_axis_name="s")

    @functools.partial(
        pl.kernel,
        out_type=(jax.ShapeDtypeStruct((NC, N_PAD, feat), jnp.float32),
                  jax.ShapeDtypeStruct((NW, ew_per), jnp.float32)),
        mesh=mesh,
        compiler_params=pltpu.CompilerParams(needs_layout_passes=False, use_tc_tiling_on_sc=False),
        scratch_types=[
            pltpu.VMEM((nchunk, K), jnp.int32),
            pltpu.VMEM((nchunk, K), jnp.int32),
            pltpu.VMEM((ew_per,), jnp.float32),
            pltpu.VMEM((ew_per,), jnp.float32),
            pltpu.VMEM((N_PAD,), jnp.float32),
            pltpu.VMEM((NBUF, K, feat), jnp.float32),
            pltpu.VMEM_SHARED((N_PAD, feat), jnp.float32),
            pltpu.VMEM_SHARED((N_PAD,), jnp.float32),
            pltpu.SemaphoreType.DMA((NBUF,)),
            pltpu.SemaphoreType.DMA((NBUF,)),
            pltpu.SemaphoreType.DMA,
        ],
    )
    def k1(h1_h, src_h, dst_h, ew_h, agg_h, nrm_h,
           src_v, dst_v, ew_v, nrm_v, dinv_v, rows_v,
           acc_sh, deg_sh, gsem, ssem, dsem):
        ci = lax.axis_index("c")
        si = lax.axis_index("s")
        w = ci * NS + si

        zero = jnp.zeros((LANES,), jnp.float32)

        def z1(i, carry):
            nrm_v[pl.ds(i * LANES, LANES)] = zero
            return carry

        lax.fori_loop(0, nper // LANES, z1, 0)
        pltpu.sync_copy(nrm_v.at[pl.ds(0, nper)],
                        deg_sh.at[pl.ds(si * nper, nper)])
        _zero_vmem(rows_v.at[0], K, feat)
        for p in range(nper // K):
            pltpu.sync_copy(rows_v.at[0],
                            acc_sh.at[pl.ds(si * nper + p * K, K)])
        plsc.subcore_barrier()

        for half in range(2):
            wd = si * 2 + half
            pltpu.async_copy(dst_h.at[wd], dst_v, gsem.at[0])
            pltpu.async_copy(ew_h.at[wd], ew_v, gsem.at[1])
            pltpu.make_async_copy(dst_h.at[wd], dst_v, gsem.at[0]).wait()
            pltpu.make_async_copy(ew_h.at[wd], ew_v, gsem.at[1]).wait()

            def fire(c, carry):
                pltpu.async_copy(ew_v.at[pl.ds(c * K, K)],
                                 deg_sh.at[dst_v.at[c]], dsem, add=True)
                return carry

            lax.fori_loop(0, nchunk, fire, 0)

            def drain(c, carry):
                pltpu.make_async_copy(ew_v.at[pl.ds(0, K)],
                                      deg_sh.at[dst_v.at[0]], dsem).wait()
                return carry

            lax.fori_loop(0, nchunk, drain, 0)
        plsc.subcore_barrier()

        pltpu.sync_copy(deg_sh.at[pl.ds(si * nper, nper)],
                        nrm_v.at[pl.ds(0, nper)])

        def binv(i, carry):
            sl = pl.ds(i * LANES, LANES)
            nrm_v[sl] = _rsqrt16(nrm_v[sl])
            return carry

        lax.fori_loop(0, nper // LANES, binv, 0)
        pltpu.sync_copy(nrm_v.at[pl.ds(0, nper)],
                        deg_sh.at[pl.ds(si * nper, nper)])
        plsc.subcore_barrier()
        pltpu.sync_copy(deg_sh, dinv_v)

        pltpu.async_copy(src_h.at[w], src_v, gsem.at[0])
        pltpu.async_copy(dst_h.at[w], dst_v, gsem.at[1])
        pltpu.async_copy(ew_h.at[w], ew_v, gsem.at[2])
        pltpu.make_async_copy(src_h.at[w], src_v, gsem.at[0]).wait()
        pltpu.make_async_copy(dst_h.at[w], dst_v, gsem.at[1]).wait()
        pltpu.make_async_copy(ew_h.at[w], ew_v, gsem.at[2]).wait()

        def bnorm(c, carry):
            for j in range(K // LANES):
                sl = pl.ds(j * LANES, LANES)
                s16 = src_v[c, sl]
                d16 = dst_v[c, sl]
                w16 = ew_v[pl.ds(c * K + j * LANES, LANES)]
                nrm = (plsc.load_gather(dinv_v, [s16]) * w16 *
                       plsc.load_gather(dinv_v, [d16]))
                nrm_v[pl.ds(c * K + j * LANES, LANES)] = nrm
            return carry

        lax.fori_loop(0, nchunk, bnorm, 0)
        pltpu.async_copy(nrm_v, nrm_h.at[w], dsem)

        _edge_phase(h1_h, src_v, dst_v, nrm_v, rows_v, acc_sh, gsem, ssem,
                    nchunk, K, feat)
        pltpu.make_async_copy(nrm_v, nrm_h.at[w], dsem).wait()
        plsc.subcore_barrier()
        pltpu.sync_copy(acc_sh.at[pl.ds(si * nper, nper)],
                        agg_h.at[ci, pl.ds(si * nper, nper)])

    return k1(h1, src3, dst3, ew2)


def _sc_layer2(hp, src3, dst3, nrm2, feat):
    _, nchunk, K = src3.shape
    nper = N_PAD // NS
    ew_per = nchunk * K
    mesh = plsc.VectorSubcoreMesh(core_axis_name="c", subcore_axis_name="s")

    @functools.partial(
        pl.kernel,
        out_type=jax.ShapeDtypeStruct((NC, N_PAD, feat), jnp.float32),
        mesh=mesh,
        compiler_params=pltpu.CompilerParams(needs_layout_passes=False, use_tc_tiling_on_sc=False),
        scratch_types=[
            pltpu.VMEM((nchunk, K), jnp.int32),
            pltpu.VMEM((nchunk, K), jnp.int32),
            pltpu.VMEM((ew_per,), jnp.float32),
            pltpu.VMEM((NBUF, K, feat), jnp.float32),
            pltpu.VMEM_SHARED((N_PAD, feat), jnp.float32),
            pltpu.SemaphoreType.DMA((NBUF,)),
            pltpu.SemaphoreType.DMA((NBUF,)),
        ],
    )
    def k2(hp_h, src_h, dst_h, nrm_h, out_h,
           src_v, dst_v, nrm_v, rows_v, acc_sh, gsem, ssem):
        ci = lax.axis_index("c")
        si = lax.axis_index("s")
        w = ci * NS + si
        pltpu.async_copy(src_h.at[w], src_v, gsem.at[0])
        pltpu.async_copy(dst_h.at[w], dst_v, gsem.at[1])
        pltpu.async_copy(nrm_h.at[w], nrm_v, gsem.at[2])
        _zero_vmem(rows_v.at[0], K, feat)
        pltpu.make_async_copy(src_h.at[w], src_v, gsem.at[0]).wait()
        pltpu.make_async_copy(dst_h.at[w], dst_v, gsem.at[1]).wait()
        pltpu.make_async_copy(nrm_h.at[w], nrm_v, gsem.at[2]).wait()
        for p in range(nper // K):
            pltpu.sync_copy(rows_v.at[0],
                            acc_sh.at[pl.ds(si * nper + p * K, K)])
        plsc.subcore_barrier()
        _edge_phase(hp_h, src_v, dst_v, nrm_v, rows_v, acc_sh, gsem, ssem,
                    nchunk, K, feat)
        plsc.subcore_barrier()
        pltpu.sync_copy(acc_sh.at[pl.ds(si * nper, nper)],
                        out_h.at[ci, pl.ds(si * nper, nper)])

    return k2(hp, src3, dst3, nrm2)


def _tc_h1(x_p, W1):
    n, d = x_p.shape
    h = W1.shape[0]

    def body(x_ref, w_ref, o_ref):
        o_ref[...] = lax.dot_general(x_ref[...], w_ref[...],
                                     (((1,), (1,)), ((), ())),
                                     preferred_element_type=jnp.float32)

    return pl.pallas_call(
        body,
        grid=(n // ROWS_BLK,),
        in_specs=[
            pl.BlockSpec((ROWS_BLK, d), lambda i: (i, 0)),
            pl.BlockSpec((h, d), lambda i: (0, 0)),
        ],
        out_specs=pl.BlockSpec((ROWS_BLK, h), lambda i: (i, 0)),
        out_shape=jax.ShapeDtypeStruct((n, h), jnp.float32),
    )(x_p, W1)


def _tc_mid(aggp, b1, W2):
    n = aggp.shape[1]
    h = aggp.shape[2]

    def body(agg_ref, b_ref, w_ref, o_ref):
        o1 = _leaky(agg_ref[0] + agg_ref[1] + b_ref[...])
        o_ref[...] = lax.dot_general(o1, w_ref[...], (((1,), (1,)), ((), ())),
                                     preferred_element_type=jnp.float32)

    return pl.pallas_call(
        body,
        grid=(n // ROWS_BLK,),
        in_specs=[
            pl.BlockSpec((NC, ROWS_BLK, h), lambda i: (0, i, 0)),
            pl.BlockSpec((1, h), lambda i: (0, 0)),
            pl.BlockSpec((h, h), lambda i: (0, 0)),
        ],
        out_specs=pl.BlockSpec((ROWS_BLK, h), lambda i: (i, 0)),
        out_shape=jax.ShapeDtypeStruct((n, h), jnp.float32),
    )(aggp, b1, W2)


def _tc_head(aggp, b2, batch2, Wl, bl):
    n = aggp.shape[1]
    h = aggp.shape[2]
    c = Wl.shape[0]
    ngrid = n // ROWS_BLK

    def body(agg_ref, b_ref, batch_ref, wl_ref, bl_ref, o_ref, acc_ref):
        i = pl.program_id(0)
        o2 = _leaky(agg_ref[0] + agg_ref[1] + b_ref[...])
        gids = lax.broadcasted_iota(jnp.int32, (G_GRAPHS, ROWS_BLK), 0)
        onehot = (gids == batch_ref[...]).astype(jnp.float32)
        aug = jnp.concatenate(
            [o2, jnp.ones((ROWS_BLK, 2 * h - h), jnp.float32)], axis=1)
        p = lax.dot_general(onehot, aug, (((1,), (0,)), ((), ())),
                            preferred_element_type=jnp.float32)

        @pl.when(i == 0)
        def _():
            acc_ref[...] = p

        @pl.when(i > 0)
        def _():
            acc_ref[...] = acc_ref[...] + p

        @pl.when(i == ngrid - 1)
        def _():
            acc = acc_ref[...]
            cnt = jnp.maximum(acc[:, h:h + 1], 1.0)
            pooled = acc[:, :h] / cnt
            logits = lax.dot_general(pooled, wl_ref[...], (((1,), (1,)), ((), ())),
                                     preferred_element_type=jnp.float32)
            logits = logits + bl_ref[...]
            m = jnp.max(logits, axis=1, keepdims=True)
            lse = m + jnp.log(jnp.sum(jnp.exp(logits - m), axis=1, keepdims=True))
            o_ref[...] = logits - lse

    return pl.pallas_call(
        body,
        grid=(ngrid,),
        in_specs=[
            pl.BlockSpec((NC, ROWS_BLK, h), lambda i: (0, i, 0)),
            pl.BlockSpec((1, h), lambda i: (0, 0)),
            pl.BlockSpec((1, ROWS_BLK), lambda i: (0, i)),
            pl.BlockSpec((c, h), lambda i: (0, 0)),
            pl.BlockSpec((1, c), lambda i: (0, 0)),
        ],
        out_specs=pl.BlockSpec((G_GRAPHS, c), lambda i: (0, 0)),
        out_shape=jax.ShapeDtypeStruct((G_GRAPHS, c), jnp.float32),
        scratch_shapes=[pltpu.VMEM((G_GRAPHS, 2 * h), jnp.float32)],
    )(aggp, b2, batch2, Wl, bl)


def kernel(x, edge_index, edge_weight, batch, W1, b1, W2, b2, Wl, bl):
    n, d = x.shape
    h = W1.shape[0]
    c = Wl.shape[0]
    e = edge_weight.shape[0]

    ew_per = e // NW
    K = 80
    nchunk = ew_per // K

    src3 = edge_index[0].reshape(NW, nchunk, K)
    dst3 = edge_index[1].reshape(NW, nchunk, K)
    ew2 = edge_weight.reshape(NW, ew_per)
    x_p = jnp.concatenate([x, jnp.zeros((N_PAD - n, d), jnp.float32)], axis=0)
    batch2 = jnp.concatenate(
        [batch, jnp.full((N_PAD - n,), G_GRAPHS, jnp.int32)]).reshape(1, N_PAD)
    b1r = b1.reshape(1, h)
    b2r = b2.reshape(1, h)
    blr = bl.reshape(1, c)

    h1 = _tc_h1(x_p, W1)
    agg1, nrm2 = _sc_layer1(h1, src3, dst3, ew2, h)
    h2 = _tc_mid(agg1, b1r, W2)
    agg2 = _sc_layer2(h2, src3, dst3, nrm2, h)
    return _tc_head(agg2, b2r, batch2, Wl, blr)


# speedup vs baseline: 1.0252x; 1.0078x over previous
import functools

import jax
import jax.numpy as jnp
from jax import lax
from jax.experimental import pallas as pl
from jax.experimental.pallas import tpu as pltpu
from jax.experimental.pallas import tpu_sc as plsc

NC = 2
NS = 16
LANES = 16
NW = NC * NS

N_PAD = 10240
ROWS_BLK = 2048
G_GRAPHS = 64

NBUF = 5
PREF = 3


def _leaky(t):
    return jnp.where(t >= 0, t, 0.01 * t)


def _rsqrt16(v):
    i = plsc.bitcast(v, jnp.int32)
    y = plsc.bitcast(jnp.int32(0x5F3759DF) - lax.shift_right_logical(i, 1),
                     jnp.float32)
    for _ in range(3):
        y = y * (1.5 - 0.5 * v * y * y)
    return jnp.where(v > 0, y, 0.0)


def _zero_vmem(ref2d, nrows, feat):
    zero = jnp.zeros((LANES,), jnp.float32)

    def zb(i, carry):
        for j in range(feat // LANES):
            ref2d[i, pl.ds(j * LANES, LANES)] = zero
        return carry

    lax.fori_loop(0, nrows, zb, 0)


def _edge_phase(hp_h, src_v, dst_v, nrm_v, rows_v, acc_sh, gsem, ssem,
                nchunk, K, feat):
    for b in range(PREF):
        pltpu.async_copy(hp_h.at[src_v.at[b]], rows_v.at[b], gsem.at[b])

    def outer(t, carry):
        for b in range(NBUF):
            c = t * NBUF + b
            pltpu.make_async_copy(
                hp_h.at[src_v.at[c]], rows_v.at[b], gsem.at[b]).wait()

            def scale(kq, cc):
                for u in range(4):
                    k = kq * 4 + u
                    wb = plsc.load_gather(
                        nrm_v, [jnp.full((LANES,), c * K + k, jnp.int32)])
                    for j in range(feat // LANES):
                        sl = pl.ds(j * LANES, LANES)
                        rows_v[b, k, sl] = rows_v[b, k, sl] * wb
                return cc

            lax.fori_loop(0, K // 4, scale, 0)
            pltpu.async_copy(rows_v.at[b], acc_sh.at[dst_v.at[c]],
                             ssem.at[b], add=True)
            cf = c + PREF
            bf = (b + PREF) % NBUF

            @pl.when(cf < nchunk)
            def _():
                @pl.when(cf >= NBUF)
                def _():
                    pltpu.make_async_copy(
                        rows_v.at[bf], acc_sh.at[dst_v.at[c]],
                        ssem.at[bf]).wait()

                pltpu.async_copy(
                    hp_h.at[src_v.at[cf]], rows_v.at[bf], gsem.at[bf])

        return carry

    lax.fori_loop(0, nchunk // NBUF, outer, 0)
    for b in range(NBUF):
        pltpu.make_async_copy(
            rows_v.at[b], acc_sh.at[dst_v.at[0]], ssem.at[b]).wait()


def _sc_layer1(h1, src3, dst3, ew2, feat):
    _, nchunk, K = src3.shape
    nper = N_PAD // NS
    ew_per = nchunk * K
    mesh = plsc.VectorSubcoreMesh(core_axis_name="c", subcore_axis_name="s")

    @functools.partial(
        pl.kernel,
        out_type=(jax.ShapeDtypeStruct((NC, N_PAD, feat), jnp.float32),
                  jax.ShapeDtypeStruct((NW, ew_per), jnp.float32)),
        mesh=mesh,
        compiler_params=pltpu.CompilerParams(needs_layout_passes=False, use_tc_tiling_on_sc=False),
        scratch_types=[
            pltpu.VMEM((nchunk, K), jnp.int32),
            pltpu.VMEM((nchunk, K), jnp.int32),
            pltpu.VMEM((ew_per,), jnp.float32),
            pltpu.VMEM((ew_per,), jnp.float32),
            pltpu.VMEM((N_PAD,), jnp.float32),
            pltpu.VMEM((NBUF, K, feat), jnp.float32),
            pltpu.VMEM_SHARED((N_PAD, feat), jnp.float32),
            pltpu.VMEM_SHARED((N_PAD,), jnp.float32),
            pltpu.SemaphoreType.DMA((NBUF,)),
            pltpu.SemaphoreType.DMA((NBUF,)),
            pltpu.SemaphoreType.DMA,
        ],
    )
    def k1(h1_h, src_h, dst_h, ew_h, agg_h, nrm_h,
           src_v, dst_v, ew_v, nrm_v, dinv_v, rows_v,
           acc_sh, deg_sh, gsem, ssem, dsem):
        ci = lax.axis_index("c")
        si = lax.axis_index("s")
        w = ci * NS + si

        zero = jnp.zeros((LANES,), jnp.float32)

        def z1(i, carry):
            nrm_v[pl.ds(i * LANES, LANES)] = zero
            return carry

        lax.fori_loop(0, nper // LANES, z1, 0)
        pltpu.async_copy(nrm_v.at[pl.ds(0, nper)],
                         deg_sh.at[pl.ds(si * nper, nper)], dsem)
        _zero_vmem(rows_v.at[0], K, feat)
        for p in range(nper // K):
            sem = gsem.at[p] if p < NBUF else ssem.at[p - NBUF]
            pltpu.async_copy(rows_v.at[0],
                             acc_sh.at[pl.ds(si * nper + p * K, K)], sem)
        pltpu.make_async_copy(nrm_v.at[pl.ds(0, nper)],
                              deg_sh.at[pl.ds(si * nper, nper)], dsem).wait()
        for p in range(nper // K):
            sem = gsem.at[p] if p < NBUF else ssem.at[p - NBUF]
            pltpu.make_async_copy(rows_v.at[0],
                                  acc_sh.at[pl.ds(si * nper + p * K, K)],
                                  sem).wait()
        plsc.subcore_barrier()

        for half in range(2):
            wd = si * 2 + half
            pltpu.async_copy(dst_h.at[wd], dst_v, gsem.at[0])
            pltpu.async_copy(ew_h.at[wd], ew_v, gsem.at[1])
            pltpu.make_async_copy(dst_h.at[wd], dst_v, gsem.at[0]).wait()
            pltpu.make_async_copy(ew_h.at[wd], ew_v, gsem.at[1]).wait()

            def fire(c, carry):
                pltpu.async_copy(ew_v.at[pl.ds(c * K, K)],
                                 deg_sh.at[dst_v.at[c]], dsem, add=True)
                return carry

            lax.fori_loop(0, nchunk, fire, 0)

            def drain(c, carry):
                pltpu.make_async_copy(ew_v.at[pl.ds(0, K)],
                                      deg_sh.at[dst_v.at[0]], dsem).wait()
                return carry

            lax.fori_loop(0, nchunk, drain, 0)
        plsc.subcore_barrier()

        pltpu.sync_copy(deg_sh.at[pl.ds(si * nper, nper)],
                        nrm_v.at[pl.ds(0, nper)])

        def binv(i, carry):
            sl = pl.ds(i * LANES, LANES)
            nrm_v[sl] = _rsqrt16(nrm_v[sl])
            return carry

        lax.fori_loop(0, nper // LANES, binv, 0)
        pltpu.sync_copy(nrm_v.at[pl.ds(0, nper)],
                        deg_sh.at[pl.ds(si * nper, nper)])
        plsc.subcore_barrier()
        pltpu.sync_copy(deg_sh, dinv_v)

        pltpu.async_copy(src_h.at[w], src_v, gsem.at[0])
        pltpu.async_copy(dst_h.at[w], dst_v, gsem.at[1])
        pltpu.async_copy(ew_h.at[w], ew_v, gsem.at[2])
        pltpu.make_async_copy(src_h.at[w], src_v, gsem.at[0]).wait()
        pltpu.make_async_copy(dst_h.at[w], dst_v, gsem.at[1]).wait()
        pltpu.make_async_copy(ew_h.at[w], ew_v, gsem.at[2]).wait()

        def bnorm(c, carry):
            for j in range(K // LANES):
                sl = pl.ds(j * LANES, LANES)
                s16 = src_v[c, sl]
                d16 = dst_v[c, sl]
                w16 = ew_v[pl.ds(c * K + j * LANES, LANES)]
                nrm = (plsc.load_gather(dinv_v, [s16]) * w16 *
                       plsc.load_gather(dinv_v, [d16]))
                nrm_v[pl.ds(c * K + j * LANES, LANES)] = nrm
            return carry

        lax.fori_loop(0, nchunk, bnorm, 0)
        pltpu.async_copy(nrm_v, nrm_h.at[w], dsem)

        _edge_phase(h1_h, src_v, dst_v, nrm_v, rows_v, acc_sh, gsem, ssem,
                    nchunk, K, feat)
        pltpu.make_async_copy(nrm_v, nrm_h.at[w], dsem).wait()
        plsc.subcore_barrier()
        pltpu.sync_copy(acc_sh.at[pl.ds(si * nper, nper)],
                        agg_h.at[ci, pl.ds(si * nper, nper)])

    return k1(h1, src3, dst3, ew2)


def _sc_layer2(hp, src3, dst3, nrm2, feat):
    _, nchunk, K = src3.shape
    nper = N_PAD // NS
    ew_per = nchunk * K
    mesh = plsc.VectorSubcoreMesh(core_axis_name="c", subcore_axis_name="s")

    @functools.partial(
        pl.kernel,
        out_type=jax.ShapeDtypeStruct((NC, N_PAD, feat), jnp.float32),
        mesh=mesh,
        compiler_params=pltpu.CompilerParams(needs_layout_passes=False, use_tc_tiling_on_sc=False),
        scratch_types=[
            pltpu.VMEM((nchunk, K), jnp.int32),
            pltpu.VMEM((nchunk, K), jnp.int32),
            pltpu.VMEM((ew_per,), jnp.float32),
            pltpu.VMEM((NBUF, K, feat), jnp.float32),
            pltpu.VMEM_SHARED((N_PAD, feat), jnp.float32),
            pltpu.SemaphoreType.DMA((NBUF,)),
            pltpu.SemaphoreType.DMA((NBUF,)),
        ],
    )
    def k2(hp_h, src_h, dst_h, nrm_h, out_h,
           src_v, dst_v, nrm_v, rows_v, acc_sh, gsem, ssem):
        ci = lax.axis_index("c")
        si = lax.axis_index("s")
        w = ci * NS + si
        pltpu.async_copy(src_h.at[w], src_v, gsem.at[0])
        pltpu.async_copy(dst_h.at[w], dst_v, gsem.at[1])
        pltpu.async_copy(nrm_h.at[w], nrm_v, gsem.at[2])
        _zero_vmem(rows_v.at[0], K, feat)
        for p in range(NBUF):
            pltpu.async_copy(rows_v.at[0],
                             acc_sh.at[pl.ds(si * nper + p * K, K)],
                             ssem.at[p])
        pltpu.make_async_copy(src_h.at[w], src_v, gsem.at[0]).wait()
        pltpu.make_async_copy(dst_h.at[w], dst_v, gsem.at[1]).wait()
        pltpu.make_async_copy(nrm_h.at[w], nrm_v, gsem.at[2]).wait()
        for p in range(NBUF):
            pltpu.make_async_copy(rows_v.at[0],
                                  acc_sh.at[pl.ds(si * nper + p * K, K)],
                                  ssem.at[p]).wait()
        for p in range(NBUF, nper // K):
            pltpu.async_copy(rows_v.at[0],
                             acc_sh.at[pl.ds(si * nper + p * K, K)],
                             ssem.at[p - NBUF])
        for p in range(NBUF, nper // K):
            pltpu.make_async_copy(rows_v.at[0],
                                  acc_sh.at[pl.ds(si * nper + p * K, K)],
                                  ssem.at[p - NBUF]).wait()
        plsc.subcore_barrier()
        _edge_phase(hp_h, src_v, dst_v, nrm_v, rows_v, acc_sh, gsem, ssem,
                    nchunk, K, feat)
        plsc.subcore_barrier()
        pltpu.sync_copy(acc_sh.at[pl.ds(si * nper, nper)],
                        out_h.at[ci, pl.ds(si * nper, nper)])

    return k2(hp, src3, dst3, nrm2)


def _tc_h1(x_p, W1):
    n, d = x_p.shape
    h = W1.shape[0]

    def body(x_ref, w_ref, o_ref):
        o_ref[...] = lax.dot_general(x_ref[...], w_ref[...],
                                     (((1,), (1,)), ((), ())),
                                     preferred_element_type=jnp.float32)

    return pl.pallas_call(
        body,
        grid=(n // ROWS_BLK,),
        in_specs=[
            pl.BlockSpec((ROWS_BLK, d), lambda i: (i, 0)),
            pl.BlockSpec((h, d), lambda i: (0, 0)),
        ],
        out_specs=pl.BlockSpec((ROWS_BLK, h), lambda i: (i, 0)),
        out_shape=jax.ShapeDtypeStruct((n, h), jnp.float32),
    )(x_p, W1)


def _tc_mid(aggp, b1, W2):
    n = aggp.shape[1]
    h = aggp.shape[2]

    def body(agg_ref, b_ref, w_ref, o_ref):
        o1 = _leaky(agg_ref[0] + agg_ref[1] + b_ref[...])
        o_ref[...] = lax.dot_general(o1, w_ref[...], (((1,), (1,)), ((), ())),
                                     preferred_element_type=jnp.float32)

    return pl.pallas_call(
        body,
        grid=(n // ROWS_BLK,),
        in_specs=[
            pl.BlockSpec((NC, ROWS_BLK, h), lambda i: (0, i, 0)),
            pl.BlockSpec((1, h), lambda i: (0, 0)),
            pl.BlockSpec((h, h), lambda i: (0, 0)),
        ],
        out_specs=pl.BlockSpec((ROWS_BLK, h), lambda i: (i, 0)),
        out_shape=jax.ShapeDtypeStruct((n, h), jnp.float32),
    )(aggp, b1, W2)


def _tc_head(aggp, b2, batch2, Wl, bl):
    n = aggp.shape[1]
    h = aggp.shape[2]
    c = Wl.shape[0]
    ngrid = n // ROWS_BLK

    def body(agg_ref, b_ref, batch_ref, wl_ref, bl_ref, o_ref, acc_ref):
        i = pl.program_id(0)
        o2 = _leaky(agg_ref[0] + agg_ref[1] + b_ref[...])
        gids = lax.broadcasted_iota(jnp.int32, (G_GRAPHS, ROWS_BLK), 0)
        onehot = (gids == batch_ref[...]).astype(jnp.float32)
        aug = jnp.concatenate(
            [o2, jnp.ones((ROWS_BLK, 2 * h - h), jnp.float32)], axis=1)
        p = lax.dot_general(onehot, aug, (((1,), (0,)), ((), ())),
                            preferred_element_type=jnp.float32)

        @pl.when(i == 0)
        def _():
            acc_ref[...] = p

        @pl.when(i > 0)
        def _():
            acc_ref[...] = acc_ref[...] + p

        @pl.when(i == ngrid - 1)
        def _():
            acc = acc_ref[...]
            cnt = jnp.maximum(acc[:, h:h + 1], 1.0)
            pooled = acc[:, :h] / cnt
            logits = lax.dot_general(pooled, wl_ref[...], (((1,), (1,)), ((), ())),
                                     preferred_element_type=jnp.float32)
            logits = logits + bl_ref[...]
            m = jnp.max(logits, axis=1, keepdims=True)
            lse = m + jnp.log(jnp.sum(jnp.exp(logits - m), axis=1, keepdims=True))
            o_ref[...] = logits - lse

    return pl.pallas_call(
        body,
        grid=(ngrid,),
        in_specs=[
            pl.BlockSpec((NC, ROWS_BLK, h), lambda i: (0, i, 0)),
            pl.BlockSpec((1, h), lambda i: (0, 0)),
            pl.BlockSpec((1, ROWS_BLK), lambda i: (0, i)),
            pl.BlockSpec((c, h), lambda i: (0, 0)),
            pl.BlockSpec((1, c), lambda i: (0, 0)),
        ],
        out_specs=pl.BlockSpec((G_GRAPHS, c), lambda i: (0, 0)),
        out_shape=jax.ShapeDtypeStruct((G_GRAPHS, c), jnp.float32),
        scratch_shapes=[pltpu.VMEM((G_GRAPHS, 2 * h), jnp.float32)],
    )(aggp, b2, batch2, Wl, bl)


def kernel(x, edge_index, edge_weight, batch, W1, b1, W2, b2, Wl, bl):
    n, d = x.shape
    h = W1.shape[0]
    c = Wl.shape[0]
    e = edge_weight.shape[0]

    ew_per = e // NW
    K = 80
    nchunk = ew_per // K

    src3 = edge_index[0].reshape(NW, nchunk, K)
    dst3 = edge_index[1].reshape(NW, nchunk, K)
    ew2 = edge_weight.reshape(NW, ew_per)
    x_p = jnp.concatenate([x, jnp.zeros((N_PAD - n, d), jnp.float32)], axis=0)
    batch2 = jnp.concatenate(
        [batch, jnp.full((N_PAD - n,), G_GRAPHS, jnp.int32)]).reshape(1, N_PAD)
    b1r = b1.reshape(1, h)
    b2r = b2.reshape(1, h)
    blr = bl.reshape(1, c)

    h1 = _tc_h1(x_p, W1)
    agg1, nrm2 = _sc_layer1(h1, src3, dst3, ew2, h)
    h2 = _tc_mid(agg1, b1r, W2)
    agg2 = _sc_layer2(h2, src3, dst3, nrm2, h)
    return _tc_head(agg2, b2r, batch2, Wl, blr)
